# Initial kernel scaffold; baseline (speedup 1.0000x reference)
import jax
import jax.numpy as jnp
from jax.experimental import pallas as pl


def kernel(features, W1, b1, Wc, bc, Wr, br):
    raise NotImplementedError("write your pallas kernel here")


# speedup vs baseline: 6.4030x; 6.4030x over previous
import math

import numpy as np
import jax
import jax.numpy as jnp
from jax.experimental import pallas as pl
from jax.experimental.pallas import tpu as pltpu

_STRIDE = 16
_IMG = 800.0
_PRE = 2000
_POST = 1000
_THR = 0.7
_MINS = 1e-3
_NEG = -1e9
_BCLIP = math.log(1000.0 / 16.0)

_MT = 512
_NMS_T = 128
_NMS_N = 2048


def _anchor_planes(H, W):
    scales = np.array([128.0, 256.0, 512.0])
    ratios = np.array([0.5, 1.0, 2.0])
    h_r = np.sqrt(ratios)
    w_r = 1.0 / h_r
    ws = (w_r[:, None] * scales[None, :]).reshape(-1)
    hs = (h_r[:, None] * scales[None, :]).reshape(-1)
    base = np.stack([-ws, -hs, ws, hs], axis=1) / 2.0
    sx = np.arange(W) * _STRIDE
    sy = np.arange(H) * _STRIDE
    yy, xx = np.meshgrid(sy, sx, indexing='ij')
    shifts = np.stack([xx, yy, xx, yy], axis=-1).reshape(-1, 4).astype(np.float32)
    anc = (shifts[:, None, :] + base[None, :, :]).astype(np.float32)
    wa = anc[:, :, 2] - anc[:, :, 0]
    ha = anc[:, :, 3] - anc[:, :, 1]
    cxa = anc[:, :, 0] + 0.5 * wa
    cya = anc[:, :, 1] + 0.5 * ha
    planes = []
    for p in (wa, ha, cxa, cya):
        planes.append(np.pad(p, ((0, 0), (0, 16 - p.shape[1]))))
    return np.concatenate(planes, axis=1).astype(np.float32)


def _head_kernel(x_ref, w1_ref, b1_ref, wcr_ref, bcr_ref, anc_ref, out_ref):
    x = x_ref[...]
    w1 = w1_ref[...]
    hid = jnp.dot(x.astype(jnp.bfloat16), w1.astype(jnp.bfloat16),
                  preferred_element_type=jnp.float32)
    hid = jnp.maximum(hid + b1_ref[...], 0.0)
    cr = jnp.dot(hid.astype(jnp.bfloat16), wcr_ref[...].astype(jnp.bfloat16),
                 preferred_element_type=jnp.float32) + bcr_ref[...]
    dx = cr[:, 0:16]
    dy = cr[:, 16:32]
    dw = jnp.minimum(cr[:, 32:48], _BCLIP)
    dh = jnp.minimum(cr[:, 48:64], _BCLIP)
    obj = cr[:, 64:80]
    anc = anc_ref[...]
    wa = anc[:, 0:16]
    ha = anc[:, 16:32]
    cxa = anc[:, 32:48]
    cya = anc[:, 48:64]
    cx = dx * wa + cxa
    cy = dy * ha + cya
    w = jnp.exp(dw) * wa
    h = jnp.exp(dh) * ha
    x1 = jnp.clip(cx - 0.5 * w, 0.0, _IMG)
    y1 = jnp.clip(cy - 0.5 * h, 0.0, _IMG)
    x2 = jnp.clip(cx + 0.5 * w, 0.0, _IMG)
    y2 = jnp.clip(cy + 0.5 * h, 0.0, _IMG)
    valid = ((x2 - x1) >= _MINS) & ((y2 - y1) >= _MINS)
    logit = jnp.where(valid, obj, _NEG)
    out_ref[:, 0:16] = x1
    out_ref[:, 16:32] = y1
    out_ref[:, 32:48] = x2
    out_ref[:, 48:64] = y2
    out_ref[:, 64:80] = logit
    out_ref[:, 80:128] = jnp.zeros_like(cr[:, 80:128])


def _nms_kernel(br_ref, bt_ref, out_ref, keepall_ref, okk_ref):
    T = _NMS_T
    N = _NMS_N
    bx1 = br_ref[:, 0:1]
    by1 = br_ref[:, 1:2]
    bx2 = br_ref[:, 2:3]
    by2 = br_ref[:, 3:4]
    area_b = (bx2 - bx1) * (by2 - by1)
    keepall_ref[...] = jnp.zeros((8, N), jnp.float32)
    iota_l = jax.lax.broadcasted_iota(jnp.int32, (1, T), 1)
    for k in range(N // T):
        s = k * T
        ax1 = bt_ref[0:1, s:s + T]
        ay1 = bt_ref[1:2, s:s + T]
        ax2 = bt_ref[2:3, s:s + T]
        ay2 = bt_ref[3:4, s:s + T]
        area_a = (ax2 - ax1) * (ay2 - ay1)
        iw = jnp.maximum(jnp.minimum(bx2, ax2) - jnp.maximum(bx1, ax1), 0.0)
        ih = jnp.maximum(jnp.minimum(by2, ay2) - jnp.maximum(by1, ay1), 0.0)
        inter = iw * ih
        over = (inter > _THR * (area_b + area_a - inter + 1e-9)).astype(jnp.float32)
        prev = keepall_ref[0:1, :]
        sup = jnp.dot(prev, over, preferred_element_type=jnp.float32)
        okk_ref[...] = over[s:s + T, :]
        kinit = jnp.where(sup > 0.5, 0.0, 1.0)

        def body(i, kloc):
            oh = (iota_l == i).astype(jnp.float32)
            srow = jnp.dot(kloc * oh, okk_ref[...],
                           preferred_element_type=jnp.float32)
            gt = (iota_l > i).astype(jnp.float32)
            return kloc * (1.0 - srow * gt)

        kloc = jax.lax.fori_loop(0, T, body, kinit)
        keepall_ref[0:1, s:s + T] = kloc
    out_ref[...] = jnp.broadcast_to(keepall_ref[0:1, :], (8, N))


def kernel(features, W1, b1, Wc, bc, Wr, br):
    B, C, H, W = features.shape
    HW = H * W
    A = Wc.shape[0]
    M = ((HW + _MT - 1) // _MT) * _MT
    G = M // _MT

    feat = jnp.transpose(features, (0, 2, 3, 1))
    xpad = jnp.pad(feat, ((0, 0), (1, 1), (1, 1), (0, 0)))
    slabs = [xpad[:, dyy:dyy + H, dxx:dxx + W, :]
             for dyy in range(3) for dxx in range(3)]
    xcol = jnp.concatenate(slabs, axis=-1).reshape(B, HW, 9 * C)
    xcol = jnp.pad(xcol, ((0, 0), (0, M - HW), (0, 0))).reshape(B * M, 9 * C)
    w1mat = jnp.transpose(W1, (2, 3, 1, 0)).reshape(9 * C, C)
    b1r = b1.reshape(1, C)
    wr_part = jnp.transpose(Wr[:, :, 0, 0].reshape(A, 4, C), (1, 0, 2))
    wr_part = jnp.pad(wr_part, ((0, 0), (0, 16 - A), (0, 0))).reshape(64, C)
    wc_part = jnp.pad(Wc[:, :, 0, 0], ((0, 16 - A), (0, 0)))
    wcr = jnp.concatenate(
        [wr_part, wc_part, jnp.zeros((48, C), jnp.float32)], axis=0).T
    br_part = jnp.pad(jnp.transpose(br.reshape(A, 4)), ((0, 0), (0, 16 - A)))
    bcr = jnp.concatenate(
        [br_part.reshape(-1), jnp.pad(bc, (0, 16 - A)),
         jnp.zeros((48,), jnp.float32)]).reshape(1, 128)
    ancp = jnp.asarray(np.pad(_anchor_planes(H, W), ((0, M - HW), (0, 0))))

    head = pl.pallas_call(
        _head_kernel,
        grid=(B, G),
        in_specs=[
            pl.BlockSpec((_MT, 9 * C), lambda bb, m: (bb * G + m, 0)),
            pl.BlockSpec((9 * C, C), lambda bb, m: (0, 0)),
            pl.BlockSpec((1, C), lambda bb, m: (0, 0)),
            pl.BlockSpec((C, 128), lambda bb, m: (0, 0)),
            pl.BlockSpec((1, 128), lambda bb, m: (0, 0)),
            pl.BlockSpec((_MT, 64), lambda bb, m: (m, 0)),
        ],
        out_specs=pl.BlockSpec((_MT, 128), lambda bb, m: (bb * G + m, 0)),
        out_shape=jax.ShapeDtypeStruct((B * M, 128), jnp.float32),
    )(xcol, w1mat, b1r, wcr, bcr, ancp)

    head = head.reshape(B, M, 128)[:, :HW]
    boxes = jnp.stack([head[..., 0:A], head[..., 16:16 + A],
                       head[..., 32:32 + A], head[..., 48:48 + A]],
                      axis=-1).reshape(B, HW * A, 4)
    logits = head[..., 64:64 + A].reshape(B, HW * A)
    scores = jnp.where(logits > _NEG / 2, jax.nn.sigmoid(logits), _NEG)

    top_s, top_i = jax.lax.top_k(scores, _PRE)
    top_b = jnp.take_along_axis(boxes, top_i[..., None], axis=1)

    bp = jnp.pad(top_b, ((0, 0), (0, _NMS_N - _PRE), (0, 0)))
    boxes_r = bp.reshape(B * _NMS_N, 4)
    boxes_t = jnp.pad(jnp.transpose(bp, (0, 2, 1)),
                      ((0, 0), (0, 4), (0, 0))).reshape(B * 8, _NMS_N)
    keep = pl.pallas_call(
        _nms_kernel,
        grid=(B,),
        in_specs=[
            pl.BlockSpec((_NMS_N, 4), lambda bb: (bb, 0)),
            pl.BlockSpec((8, _NMS_N), lambda bb: (bb, 0)),
        ],
        out_specs=pl.BlockSpec((8, _NMS_N), lambda bb: (bb, 0)),
        out_shape=jax.ShapeDtypeStruct((B * 8, _NMS_N), jnp.float32),
        scratch_shapes=[
            pltpu.VMEM((8, _NMS_N), jnp.float32),
            pltpu.VMEM((_NMS_T, _NMS_T), jnp.float32),
        ],
    )(boxes_r, boxes_t)
    keep = keep.reshape(B, 8, _NMS_N)[:, 0, :_PRE] > 0.5

    kept_s = jnp.where(keep, top_s, _NEG)
    f_s, f_i = jax.lax.top_k(kept_s, _POST)
    f_b = jnp.take_along_axis(top_b, f_i[..., None], axis=1)
    mask = f_s > _NEG / 2
    out_boxes = jnp.where(mask[..., None], f_b, 0.0)
    out_scores = jnp.where(mask, f_s, 0.0)
    return out_boxes, out_scores


# speedup vs baseline: 8.5482x; 1.3350x over previous
import math

import numpy as np
import jax
import jax.numpy as jnp
from jax.experimental import pallas as pl
from jax.experimental.pallas import tpu as pltpu

_STRIDE = 16
_IMG = 800.0
_PRE = 2000
_POST = 1000
_THR = 0.7
_MINS = 1e-3
_NEG = -1e9
_BCLIP = math.log(1000.0 / 16.0)

_MT = 512
_NMS_T = 128
_NMS_N = 2048


def _anchor_planes(H, W):
    scales = np.array([128.0, 256.0, 512.0])
    ratios = np.array([0.5, 1.0, 2.0])
    h_r = np.sqrt(ratios)
    w_r = 1.0 / h_r
    ws = (w_r[:, None] * scales[None, :]).reshape(-1)
    hs = (h_r[:, None] * scales[None, :]).reshape(-1)
    base = np.stack([-ws, -hs, ws, hs], axis=1) / 2.0
    sx = np.arange(W) * _STRIDE
    sy = np.arange(H) * _STRIDE
    yy, xx = np.meshgrid(sy, sx, indexing='ij')
    shifts = np.stack([xx, yy, xx, yy], axis=-1).reshape(-1, 4).astype(np.float32)
    anc = (shifts[:, None, :] + base[None, :, :]).astype(np.float32)
    wa = anc[:, :, 2] - anc[:, :, 0]
    ha = anc[:, :, 3] - anc[:, :, 1]
    cxa = anc[:, :, 0] + 0.5 * wa
    cya = anc[:, :, 1] + 0.5 * ha
    planes = []
    for p in (wa, ha, cxa, cya):
        planes.append(np.pad(p, ((0, 0), (0, 16 - p.shape[1]))))
    return np.concatenate(planes, axis=1).astype(np.float32)


def _head_kernel(x_ref, w1_ref, b1_ref, wcr_ref, bcr_ref, anc_ref, out_ref,
                 hid_ref):
    P = hid_ref.shape[0]
    C = w1_ref.shape[1]
    hid_ref[...] = jnp.zeros_like(hid_ref)
    for t, (dyy, dxx) in enumerate((a, b) for a in range(3) for b in range(3)):
        off = dyy * 52 + dxx
        xs = x_ref[off:P, :].astype(jnp.bfloat16)
        wt = w1_ref[t * C:(t + 1) * C, :].astype(jnp.bfloat16)
        hid_ref[0:P - off, :] += jnp.dot(xs, wt,
                                         preferred_element_type=jnp.float32)
    hid = jnp.maximum(hid_ref[...] + b1_ref[...], 0.0)
    cr = jnp.dot(hid.astype(jnp.bfloat16), wcr_ref[...].astype(jnp.bfloat16),
                 preferred_element_type=jnp.float32) + bcr_ref[...]
    dx = cr[:, 0:16]
    dy = cr[:, 16:32]
    dw = jnp.minimum(cr[:, 32:48], _BCLIP)
    dh = jnp.minimum(cr[:, 48:64], _BCLIP)
    obj = cr[:, 64:80]
    anc = anc_ref[...]
    wa = anc[:, 0:16]
    ha = anc[:, 16:32]
    cxa = anc[:, 32:48]
    cya = anc[:, 48:64]
    cx = dx * wa + cxa
    cy = dy * ha + cya
    w = jnp.exp(dw) * wa
    h = jnp.exp(dh) * ha
    x1 = jnp.clip(cx - 0.5 * w, 0.0, _IMG)
    y1 = jnp.clip(cy - 0.5 * h, 0.0, _IMG)
    x2 = jnp.clip(cx + 0.5 * w, 0.0, _IMG)
    y2 = jnp.clip(cy + 0.5 * h, 0.0, _IMG)
    valid = ((x2 - x1) >= _MINS) & ((y2 - y1) >= _MINS)
    logit = jnp.where(valid, obj, _NEG)
    out_ref[:, 0:16] = x1
    out_ref[:, 16:32] = y1
    out_ref[:, 32:48] = x2
    out_ref[:, 48:64] = y2
    out_ref[:, 64:80] = logit
    out_ref[:, 80:128] = jnp.zeros_like(cr[:, 80:128])


def _nms_kernel(br_ref, bt_ref, out_ref, keepall_ref, okk_ref):
    T = _NMS_T
    N = _NMS_N
    bx1 = br_ref[:, 0:1]
    by1 = br_ref[:, 1:2]
    bx2 = br_ref[:, 2:3]
    by2 = br_ref[:, 3:4]
    area_b = (bx2 - bx1) * (by2 - by1)
    keepall_ref[...] = jnp.zeros((8, N), jnp.float32)
    iota_l = jax.lax.broadcasted_iota(jnp.int32, (1, T), 1)
    for k in range(N // T):
        s = k * T
        ax1 = bt_ref[0:1, s:s + T]
        ay1 = bt_ref[1:2, s:s + T]
        ax2 = bt_ref[2:3, s:s + T]
        ay2 = bt_ref[3:4, s:s + T]
        area_a = (ax2 - ax1) * (ay2 - ay1)
        iw = jnp.maximum(jnp.minimum(bx2, ax2) - jnp.maximum(bx1, ax1), 0.0)
        ih = jnp.maximum(jnp.minimum(by2, ay2) - jnp.maximum(by1, ay1), 0.0)
        inter = iw * ih
        over = (inter > _THR * (area_b + area_a - inter + 1e-9)).astype(jnp.float32)
        prev = keepall_ref[0:1, :]
        sup = jnp.dot(prev, over, preferred_element_type=jnp.float32)
        okk_ref[...] = over[s:s + T, :]
        kinit = jnp.where(sup > 0.5, 0.0, 1.0)

        def body(i, kloc):
            oh = (iota_l == i).astype(jnp.float32)
            srow = jnp.dot(kloc * oh, okk_ref[...],
                           preferred_element_type=jnp.float32)
            gt = (iota_l > i).astype(jnp.float32)
            return kloc * (1.0 - srow * gt)

        kloc = jax.lax.fori_loop(0, T, body, kinit)
        keepall_ref[0:1, s:s + T] = kloc
    out_ref[...] = jnp.broadcast_to(keepall_ref[0:1, :], (8, N))


def kernel(features, W1, b1, Wc, bc, Wr, br):
    B, C, H, W = features.shape
    HW = H * W
    A = Wc.shape[0]
    Hp, Wp = H + 2, W + 2
    P = Hp * Wp

    feat = jnp.transpose(features, (0, 2, 3, 1))
    xpad = jnp.pad(feat, ((0, 0), (1, 1), (1, 1), (0, 0)))
    xflat = xpad.reshape(B * P, C)
    w1mat = jnp.transpose(W1, (2, 3, 1, 0)).reshape(9 * C, C)
    b1r = b1.reshape(1, C)
    wr_part = jnp.transpose(Wr[:, :, 0, 0].reshape(A, 4, C), (1, 0, 2))
    wr_part = jnp.pad(wr_part, ((0, 0), (0, 16 - A), (0, 0))).reshape(64, C)
    wc_part = jnp.pad(Wc[:, :, 0, 0], ((0, 16 - A), (0, 0)))
    wcr = jnp.concatenate(
        [wr_part, wc_part, jnp.zeros((48, C), jnp.float32)], axis=0).T
    br_part = jnp.pad(jnp.transpose(br.reshape(A, 4)), ((0, 0), (0, 16 - A)))
    bcr = jnp.concatenate(
        [br_part.reshape(-1), jnp.pad(bc, (0, 16 - A)),
         jnp.zeros((48,), jnp.float32)]).reshape(1, 128)
    anc52 = np.pad(_anchor_planes(H, W).reshape(H, W, 64),
                   ((0, 2), (0, 2), (0, 0))).reshape(P, 64)
    ancp = jnp.asarray(anc52)

    head = pl.pallas_call(
        _head_kernel,
        grid=(B,),
        in_specs=[
            pl.BlockSpec((P, C), lambda bb: (bb, 0)),
            pl.BlockSpec((9 * C, C), lambda bb: (0, 0)),
            pl.BlockSpec((1, C), lambda bb: (0, 0)),
            pl.BlockSpec((C, 128), lambda bb: (0, 0)),
            pl.BlockSpec((1, 128), lambda bb: (0, 0)),
            pl.BlockSpec((P, 64), lambda bb: (0, 0)),
        ],
        out_specs=pl.BlockSpec((P, 128), lambda bb: (bb, 0)),
        out_shape=jax.ShapeDtypeStruct((B * P, 128), jnp.float32),
        scratch_shapes=[pltpu.VMEM((P, C), jnp.float32)],
    )(xflat, w1mat, b1r, wcr, bcr, ancp)

    head = head.reshape(B, Hp, Wp, 128)[:, :H, :W].reshape(B, HW, 128)
    boxes = jnp.stack([head[..., 0:A], head[..., 16:16 + A],
                       head[..., 32:32 + A], head[..., 48:48 + A]],
                      axis=-1).reshape(B, HW * A, 4)
    logits = head[..., 64:64 + A].reshape(B, HW * A)
    scores = jnp.where(logits > _NEG / 2, jax.nn.sigmoid(logits), _NEG)

    top_s, top_i = jax.lax.top_k(scores, _PRE)
    top_b = jnp.take_along_axis(boxes, top_i[..., None], axis=1)

    bp = jnp.pad(top_b, ((0, 0), (0, _NMS_N - _PRE), (0, 0)))
    boxes_r = bp.reshape(B * _NMS_N, 4)
    boxes_t = jnp.pad(jnp.transpose(bp, (0, 2, 1)),
                      ((0, 0), (0, 4), (0, 0))).reshape(B * 8, _NMS_N)
    keep = pl.pallas_call(
        _nms_kernel,
        grid=(B,),
        in_specs=[
            pl.BlockSpec((_NMS_N, 4), lambda bb: (bb, 0)),
            pl.BlockSpec((8, _NMS_N), lambda bb: (bb, 0)),
        ],
        out_specs=pl.BlockSpec((8, _NMS_N), lambda bb: (bb, 0)),
        out_shape=jax.ShapeDtypeStruct((B * 8, _NMS_N), jnp.float32),
        scratch_shapes=[
            pltpu.VMEM((8, _NMS_N), jnp.float32),
            pltpu.VMEM((_NMS_T, _NMS_T), jnp.float32),
        ],
    )(boxes_r, boxes_t)
    keep = keep.reshape(B, 8, _NMS_N)[:, 0, :_PRE] > 0.5

    kept_s = jnp.where(keep, top_s, _NEG)
    f_s, f_i = jax.lax.top_k(kept_s, _POST)
    f_b = jnp.take_along_axis(top_b, f_i[..., None], axis=1)
    mask = f_s > _NEG / 2
    out_boxes = jnp.where(mask[..., None], f_b, 0.0)
    out_scores = jnp.where(mask, f_s, 0.0)
    return out_boxes, out_scores
